# baseline (speedup 1.0000x reference)
import jax
import jax.numpy as jnp
from jax.experimental import pallas as pl
from jax.experimental.pallas import tpu as pltpu

_VOCABS = (40001, 30001, 28001)
_OFFSETS = (0, 40001, 70002)
_F = 3

_BB = 256


def _gather_body(bb, v, d):
    def body(i0_ref, i1_ref, i2_ref, table_ref, o_ref):
        t3 = table_ref.reshape(v, 1, d)
        base = pl.program_id(0) * bb
        for bi in range(bb):
            for f, iref in enumerate((i0_ref, i1_ref, i2_ref)):
                o_ref[bi, f] = t3[iref[base + bi], 0]
    return body


def kernel(table, user_id, item_id, cate_id):
    v, d = table.shape
    ids = [
        jnp.clip(raw.astype(jnp.int32), 0, vocab - 1) + off
        for raw, vocab, off in zip(
            (user_id, item_id, cate_id), _VOCABS, _OFFSETS)
    ]
    b = user_id.shape[0]

    out = pl.pallas_call(
        _gather_body(_BB, v, d),
        out_shape=jax.ShapeDtypeStruct((b, _F, d), table.dtype),
        grid_spec=pltpu.PrefetchScalarGridSpec(
            num_scalar_prefetch=3,
            grid=(b // _BB,),
            in_specs=[pl.BlockSpec((v, d), lambda i, p0, p1, p2: (0, 0))],
            out_specs=pl.BlockSpec(
                (_BB, _F, d), lambda i, p0, p1, p2: (i, 0, 0)),
        ),
        compiler_params=pltpu.CompilerParams(
            dimension_semantics=("parallel",),
        ),
    )(*ids, table)
    return out


# speedup vs baseline: 1.0916x; 1.0916x over previous
import jax
import jax.numpy as jnp
from jax.experimental import pallas as pl
from jax.experimental.pallas import tpu as pltpu

_VOCABS = (40001, 30001, 28001)
_OFFSETS = (0, 40001, 70002)
_F = 3

_BB = 512


def _gather_body(bb, v, d):
    def body(i0_ref, i1_ref, i2_ref, table_ref, o_ref):
        t3 = table_ref.reshape(v, 1, d)
        base = pl.program_id(0) * bb
        for bi in range(bb):
            for f, iref in enumerate((i0_ref, i1_ref, i2_ref)):
                o_ref[bi, f] = t3[iref[base + bi], 0]
    return body


def kernel(table, user_id, item_id, cate_id):
    v, d = table.shape
    ids = [
        jnp.clip(raw.astype(jnp.int32), 0, vocab - 1) + off
        for raw, vocab, off in zip(
            (user_id, item_id, cate_id), _VOCABS, _OFFSETS)
    ]
    b = user_id.shape[0]

    out = pl.pallas_call(
        _gather_body(_BB, v, d),
        out_shape=jax.ShapeDtypeStruct((b, _F, d), table.dtype),
        grid_spec=pltpu.PrefetchScalarGridSpec(
            num_scalar_prefetch=3,
            grid=(b // _BB,),
            in_specs=[pl.BlockSpec((v, d), lambda i, p0, p1, p2: (0, 0))],
            out_specs=pl.BlockSpec(
                (_BB, _F, d), lambda i, p0, p1, p2: (i, 0, 0)),
        ),
        compiler_params=pltpu.CompilerParams(
            dimension_semantics=("parallel",),
        ),
    )(*ids, table)
    return out
